# baseline (speedup 1.0000x reference)
import math

import jax
import jax.numpy as jnp
from jax.experimental import pallas as pl

B_, N_, E_, D_, NR, NL, TAU, M_ = 4, 2048, 16384, 64, 500, 3, 0.1, 20


def _linear(x, W, b):
    return x @ W + b


def _ln(x, g, b):
    m = x.mean(-1, keepdims=True)
    v = ((x - m) ** 2).mean(-1, keepdims=True)
    return (x - m) / jnp.sqrt(v + 1e-5) * g + b


def _gather(h, idx):
    return jnp.take_along_axis(h, idx[:, :, None], axis=1)


def _identity_kernel(x_ref, o_ref):
    o_ref[...] = x_ref[...]


def kernel(edge_index, rels, dists, query_rels, edge_conf_mask, edge_mask, node_mask,
           scores, conf_B, conf_W, conf_b, rel_table, lre_beta_W, lre_beta_b,
           lre_msg_W, lre_msg_b, lre_upd_W, lre_upd_b, lre_ln_g, lre_ln_b,
           dist_table, sfe_msg_W, sfe_msg_b, sfe_upd_W, sfe_upd_b, att_W, att_b,
           Wq, bq, Wk, bk, Wv, bv, fmr_ln_g, fmr_ln_b):
    B, N = node_mask.shape
    E = rels.shape[1]
    D = rel_table.shape[1]
    src = edge_index[:, 0, :]
    dst = edge_index[:, 1, :]
    bidx = jnp.broadcast_to(jnp.arange(B)[:, None], (B, E))
    ecm = edge_conf_mask
    emf = edge_mask[:, :, None].astype(jnp.float32)
    s3 = jnp.where(ecm[:, :, None], scores[:, :, None], 0.0)
    xp = 2 * math.pi * s3 @ conf_B
    conf = _linear(jnp.concatenate([jnp.cos(xp), jnp.sin(xp)], -1), conf_W, conf_b)
    rq = rel_table[query_rels]
    h_r = rel_table[rels]
    h_init = jnp.zeros((B, N, D)).at[:, 0, :].set(1.0)
    h = h_init
    h_init_src = _gather(h_init, src)
    beta = jax.nn.sigmoid(_linear(h_r + rq[:, None, :], lre_beta_W, lre_beta_b))
    gate_known = jax.nn.sigmoid((scores[:, :, None] - beta) / TAU)
    gate = jnp.where(ecm[:, :, None], gate_known, 0.5)
    ctx = []
    for k in range(NL):
        h_src = _gather(h, src)
        msg_in = jnp.concatenate([h_src * h_r, h_src, h_init_src, h_r, conf], -1)
        raw = jax.nn.relu(_linear(msg_in, lre_msg_W[k], lre_msg_b[k]))
        wm = gate * raw * emf
        aggr = jnp.zeros((B, N, D)).at[bidx, dst].add(wm)
        h = _ln(h + _linear(aggr, lre_upd_W[k], lre_upd_b[k]), lre_ln_g, lre_ln_b)
        ctx.append(h[:, 0, :])
    H_ctx = jnp.stack(ctx, 1)
    dist_emb = dist_table[jnp.clip(dists, 0, 9)]
    noise = jax.random.normal(jax.random.key(42), dist_emb.shape) * 0.1
    h2 = dist_emb + noise
    for k in range(NL):
        h_src = _gather(h2, src)
        dist_src = _gather(dist_emb, src)
        msg = jax.nn.relu(_linear(jnp.concatenate([h_src * h_r, h_src, dist_src, h_r, conf], -1), sfe_msg_W[k], sfe_msg_b[k]))
        msg = msg * emf
        aggr = jnp.zeros((B, N, D)).at[bidx, dst].add(msg)
        h2 = _linear(aggr, sfe_upd_W[k], sfe_upd_b[k]) + h2
    h2 = h2 * node_mask[:, :, None].astype(jnp.float32)
    t_state = h2[:, 0, :]
    att_in = jnp.concatenate([h2, jnp.broadcast_to(rq[:, None, :], (B, N, D))], -1)
    att = jax.nn.leaky_relu(_linear(att_in, att_W, att_b)).squeeze(-1)
    att = jnp.where(node_mask, att, -1e9)
    alpha = jax.nn.softmax(att, axis=1)
    tv, ti = jax.lax.top_k(alpha, M_)
    H_evd = jnp.take_along_axis(h2, ti[:, :, None], axis=1) * tv[:, :, None]
    x = jnp.concatenate([H_ctx, H_evd], 1)
    Nt = x.shape[1]
    H = 4
    Dh = D // H

    def split(t):
        return t.reshape(B, Nt, H, Dh).transpose(0, 2, 1, 3)

    q = split(_linear(x, Wq, bq))
    k_ = split(_linear(x, Wk, bk))
    v = split(_linear(x, Wv, bv))

    def nrm(t):
        return t / jnp.maximum(jnp.linalg.norm(t, axis=-1, keepdims=True), 1e-12)

    q = nrm(q)
    k_ = nrm(k_)
    kvs = jnp.einsum('bhnd,bhnD->bhdD', k_, v)
    num = jnp.einsum('bhnd,bhdD->bhnD', q, kvs) + v.sum(2, keepdims=True) + v * Nt
    den = jnp.einsum('bhnd,bhd->bhn', q, k_.sum(2)) + Nt + Nt
    out = num / den[..., None]
    out = out.transpose(0, 2, 1, 3).reshape(B, Nt, D)
    y = _ln(x + out, fmr_ln_g, fmr_ln_b)
    res = y.mean(1) + t_state
    return pl.pallas_call(
        _identity_kernel,
        out_shape=jax.ShapeDtypeStruct(res.shape, res.dtype),
    )(res)


# speedup vs baseline: 2.6815x; 2.6815x over previous
import functools
import math

import jax
import jax.numpy as jnp
from jax import lax
from jax.experimental import pallas as pl
from jax.experimental.pallas import tpu as pltpu

B_, N_, E_, D_ = 4, 2048, 16384, 64
NR, NL, TAU, M_ = 500, 3, 0.1, 20
BE = B_ * E_
BN = B_ * N_
EC = 2048
NEC = BE // EC
CPB = E_ // EC


def _pre_body(scores_ref, ecm_ref, hr_ref, qr_ref, confB_ref, confW_ref,
              confb_ref, rel_ref, betaW_ref, betab_ref, conf_ref, gate_ref):
    b = pl.program_id(0) // CPB
    s = scores_ref[...]
    m = ecm_ref[...]
    s3 = s * m
    xp = (2.0 * math.pi) * s3 * confB_ref[...]
    cs = jnp.concatenate([jnp.cos(xp), jnp.sin(xp)], axis=1)
    conf_ref[...] = cs @ confW_ref[...] + confb_ref[...]
    rtb = rel_ref[...] @ betaW_ref[...]
    qr = qr_ref[...]
    i500 = lax.broadcasted_iota(jnp.int32, (B_, NR), 1)
    qoh = (qr == i500).astype(jnp.float32)
    rqbw = qoh @ rtb
    i4 = lax.broadcasted_iota(jnp.int32, (B_, 1), 0)
    rqbw_b = jnp.sum(jnp.where(i4 == b, rqbw, 0.0), axis=0, keepdims=True)
    beta = jax.nn.sigmoid(hr_ref[...] @ betaW_ref[...] + rqbw_b + betab_ref[...])
    gate = m * jax.nn.sigmoid((s - beta) / TAU) + (1.0 - m) * 0.5
    gate_ref[...] = gate


def _precompute(scores_f, ecm_f, h_r, query_rels, conf_B, conf_W, conf_b,
                rel_table, beta_W, beta_b):
    full = lambda shape: pl.BlockSpec(shape, lambda i: (0, 0))
    chunk = lambda w: pl.BlockSpec((EC, w), lambda i: (i, 0))
    return pl.pallas_call(
        _pre_body,
        grid=(NEC,),
        in_specs=[chunk(1), chunk(1), chunk(D_), full((B_, 1)),
                  full((1, D_ // 2)), full((D_, D_)), full((1, D_)),
                  full((NR, D_)), full((D_, 1)), full((1, 1))],
        out_specs=[chunk(D_), chunk(1)],
        out_shape=[jax.ShapeDtypeStruct((BE, D_), jnp.float32),
                   jax.ShapeDtypeStruct((BE, 1), jnp.float32)],
    )(scores_f, ecm_f, h_r, query_rels, conf_B, conf_W, conf_b,
      rel_table, beta_W, beta_b)


def _lre_msg_body(hs_ref, hr_ref, cf_ref, gate_ref, src0_ref, Wc_ref,
                  csum_ref, bk_ref, wm_ref):
    hs = hs_ref[...]
    hr = hr_ref[...]
    x = jnp.concatenate([hs * hr, hs, hr, cf_ref[...]], axis=1)
    raw = x @ Wc_ref[...] + src0_ref[...] * csum_ref[...] + bk_ref[...]
    wm_ref[...] = gate_ref[...] * jnp.maximum(raw, 0.0)


def _lre_msg(h_src, h_r, conf, gate, src0, Wc, csum3, bk):
    full = lambda shape: pl.BlockSpec(shape, lambda i: (0, 0))
    chunk = lambda w: pl.BlockSpec((EC, w), lambda i: (i, 0))
    return pl.pallas_call(
        _lre_msg_body,
        grid=(NEC,),
        in_specs=[chunk(D_), chunk(D_), chunk(D_), chunk(1), chunk(1),
                  full((4 * D_, D_)), full((1, D_)), full((1, D_))],
        out_specs=chunk(D_),
        out_shape=jax.ShapeDtypeStruct((BE, D_), jnp.float32),
    )(h_src, h_r, conf, gate, src0, Wc, csum3, bk)


def _sfe_msg_body(hs_ref, hr_ref, ds_ref, cf_ref, Wc_ref, bk_ref, wm_ref):
    hs = hs_ref[...]
    hr = hr_ref[...]
    x = jnp.concatenate([hs * hr, hs, ds_ref[...], hr, cf_ref[...]], axis=1)
    wm_ref[...] = jnp.maximum(x @ Wc_ref[...] + bk_ref[...], 0.0)


def _sfe_msg(h_src, h_r, dist_src, conf, Wc, bk):
    full = lambda shape: pl.BlockSpec(shape, lambda i: (0, 0))
    chunk = lambda w: pl.BlockSpec((EC, w), lambda i: (i, 0))
    return pl.pallas_call(
        _sfe_msg_body,
        grid=(NEC,),
        in_specs=[chunk(D_), chunk(D_), chunk(D_), chunk(D_),
                  full((5 * D_, D_)), full((1, D_))],
        out_specs=chunk(D_),
        out_shape=jax.ShapeDtypeStruct((BE, D_), jnp.float32),
    )(h_src, h_r, dist_src, conf, Wc, bk)


def _ln_rows(x, g, b):
    m = jnp.mean(x, axis=1, keepdims=True)
    v = jnp.mean((x - m) ** 2, axis=1, keepdims=True)
    return (x - m) / jnp.sqrt(v + 1e-5) * g + b


def _lre_upd_body(p0_ref, p1_ref, h_ref, W_ref, b_ref, g_ref, lb_ref, o_ref):
    aggr = p0_ref[...] + p1_ref[...]
    o_ref[...] = _ln_rows(h_ref[...] + aggr @ W_ref[...] + b_ref[...],
                          g_ref[...], lb_ref[...])


def _lre_upd(p0, p1, h, W, b, g, lb):
    full = lambda shape: pl.BlockSpec(shape, lambda: (0, 0))
    return pl.pallas_call(
        _lre_upd_body,
        in_specs=[full((BN, D_)), full((BN, D_)), full((BN, D_)),
                  full((D_, D_)), full((1, D_)), full((1, D_)), full((1, D_))],
        out_specs=full((BN, D_)),
        out_shape=jax.ShapeDtypeStruct((BN, D_), jnp.float32),
    )(p0, p1, h, W, b, g, lb)


def _sfe_upd_body(p0_ref, p1_ref, h_ref, W_ref, b_ref, o_ref):
    aggr = p0_ref[...] + p1_ref[...]
    o_ref[...] = h_ref[...] + aggr @ W_ref[...] + b_ref[...]


def _sfe_upd(p0, p1, h, W, b):
    full = lambda shape: pl.BlockSpec(shape, lambda: (0, 0))
    return pl.pallas_call(
        _sfe_upd_body,
        in_specs=[full((BN, D_)), full((BN, D_)), full((BN, D_)),
                  full((D_, D_)), full((1, D_))],
        out_specs=full((BN, D_)),
        out_shape=jax.ShapeDtypeStruct((BN, D_), jnp.float32),
    )(p0, p1, h, W, b)


def _add_body(a_ref, b_ref, o_ref):
    o_ref[...] = a_ref[...] + b_ref[...]


def _add2(a, b):
    full = pl.BlockSpec((BN, D_), lambda: (0, 0))
    return pl.pallas_call(
        _add_body,
        in_specs=[full, full],
        out_specs=full,
        out_shape=jax.ShapeDtypeStruct((BN, D_), jnp.float32),
    )(a, b)


def _finale_body(ctx_ref, h2f_ref, qr_ref, rel_ref,
                 attW1_ref, attW2_ref, attb_ref, Wq_ref, bq_ref, Wk_ref,
                 bk_ref, Wv_ref, bv_ref, g_ref, lb_ref, o_ref):
    b = pl.program_id(0)
    h2f = h2f_ref[...]
    t_state = h2f[0:1, :]
    qr = qr_ref[...]
    i4 = lax.broadcasted_iota(jnp.int32, (B_, 1), 0)
    qr_b = jnp.sum(jnp.where(i4 == b, qr, 0), axis=0, keepdims=True)
    i500c = lax.broadcasted_iota(jnp.int32, (1, NR), 1)
    qoh = (i500c == qr_b).astype(jnp.float32)
    rq = qoh @ rel_ref[...]
    att = h2f @ attW1_ref[...] + (rq @ attW2_ref[...] + attb_ref[...])
    att = jnp.where(att >= 0.0, att, 0.01 * att)
    amax = jnp.max(att, axis=0, keepdims=True)
    ex = jnp.exp(att - amax)
    alpha = ex / jnp.sum(ex, axis=0, keepdims=True)
    iota = lax.broadcasted_iota(jnp.int32, (N_, 1), 0)
    acur = alpha
    rows = []
    for _ in range(M_):
        v = jnp.max(acur, axis=0, keepdims=True)
        eq = acur == v
        fidx = jnp.min(jnp.where(eq, iota, N_), axis=0, keepdims=True)
        sel = iota == fidx
        ohf = sel.astype(jnp.float32)
        rows.append(jnp.sum(ohf * h2f, axis=0, keepdims=True) * v)
        acur = jnp.where(sel, -1.0, acur)
    ctx = ctx_ref[...].reshape(8, D_)[0:NL, :]
    x = jnp.concatenate([ctx] + rows, axis=0)
    Nt = NL + M_
    hd = lax.broadcasted_iota(jnp.int32, (D_, D_), 0) // 16
    hD = lax.broadcasted_iota(jnp.int32, (D_, D_), 1) // 16
    blockones = (hd == hD).astype(jnp.float32)
    q = x @ Wq_ref[...] + bq_ref[...]
    k_ = x @ Wk_ref[...] + bk_ref[...]
    v_ = x @ Wv_ref[...] + bv_ref[...]

    def nrmh(t):
        ssum = (t * t) @ blockones
        return t / jnp.maximum(jnp.sqrt(ssum), 1e-12)

    q = nrmh(q)
    k_ = nrmh(k_)
    KtV = lax.dot_general(k_, v_, (((0,), (0,)), ((), ())))
    kvs = KtV * blockones
    vsum = jnp.sum(v_, axis=0, keepdims=True)
    ksum = jnp.sum(k_, axis=0, keepdims=True)
    num = q @ kvs + vsum + v_ * float(Nt)
    den = (q * ksum) @ blockones + float(2 * Nt)
    out = num / den
    y = _ln_rows(x + out, g_ref[...], lb_ref[...])
    res = jnp.mean(y, axis=0, keepdims=True) + t_state
    o_ref[...] = jnp.concatenate(
        [res, jnp.zeros((7, D_), jnp.float32)], axis=0).reshape(1, 8, D_)


def _finale(ctx_all, h2f, query_rels, rel_table, attW1, attW2, attb,
            Wq, bq, Wk, bk, Wv, bv, g, lb):
    full = lambda shape: pl.BlockSpec(shape, lambda b: tuple(0 for _ in shape))
    return pl.pallas_call(
        _finale_body,
        grid=(B_,),
        in_specs=[pl.BlockSpec((1, 8, D_), lambda b: (b, 0, 0)),
                  pl.BlockSpec((N_, D_), lambda b: (b, 0)),
                  full((B_, 1)), full((NR, D_)),
                  full((D_, 1)), full((D_, 1)), full((1, 1)),
                  full((D_, D_)), full((1, D_)), full((D_, D_)), full((1, D_)),
                  full((D_, D_)), full((1, D_)), full((1, D_)), full((1, D_))],
        out_specs=pl.BlockSpec((1, 8, D_), lambda b: (b, 0, 0)),
        out_shape=jax.ShapeDtypeStruct((B_, 8, D_), jnp.float32),
    )(ctx_all, h2f, query_rels, rel_table, attW1, attW2, attb,
      Wq, bq, Wk, bk, Wv, bv, g, lb)[:, 0, :]


def _gather_rows(table, idx):
    return table[idx]


def _scatter_add(vals, idx):
    return jnp.zeros((BN, D_), jnp.float32).at[idx].add(vals)


def kernel(edge_index, rels, dists, query_rels, edge_conf_mask, edge_mask,
           node_mask, scores, conf_B, conf_W, conf_b, rel_table, lre_beta_W,
           lre_beta_b, lre_msg_W, lre_msg_b, lre_upd_W, lre_upd_b, lre_ln_g,
           lre_ln_b, dist_table, sfe_msg_W, sfe_msg_b, sfe_upd_W, sfe_upd_b,
           att_W, att_b, Wq, bq, Wk, bk, Wv, bv, fmr_ln_g, fmr_ln_b):
    f32 = jnp.float32
    src = edge_index[:, 0, :].astype(jnp.int32).reshape(BE)
    dst = edge_index[:, 1, :].astype(jnp.int32).reshape(BE)
    boff = jnp.repeat(jnp.arange(B_, dtype=jnp.int32) * N_, E_)
    srcg = src + boff
    dstg = dst + boff
    rels_f = rels.astype(jnp.int32).reshape(BE)
    dclip = jnp.clip(dists, 0, 9).astype(jnp.int32).reshape(BN)
    scores_f = scores.astype(f32).reshape(BE, 1)
    ecm_f = edge_conf_mask.astype(f32).reshape(BE, 1)
    src0 = (src == 0).astype(f32).reshape(BE, 1)
    qr2 = query_rels.astype(jnp.int32).reshape(B_, 1)
    conf_b2 = conf_b.reshape(1, D_)
    beta_b2 = lre_beta_b.reshape(1, 1)
    zeros_bn = jnp.zeros((BN, D_), f32)

    h_r = _gather_rows(rel_table, rels_f)
    dist_emb = _gather_rows(dist_table, dclip)
    dist_src = _gather_rows(dist_emb, srcg)

    conf, gate = _precompute(scores_f, ecm_f, h_r, qr2, conf_B, conf_W,
                             conf_b2, rel_table, lre_beta_W, beta_b2)

    lre_g = lre_ln_g.reshape(1, D_)
    lre_b = lre_ln_b.reshape(1, D_)
    h = jnp.zeros((BN, D_), f32).at[jnp.arange(B_) * N_].set(1.0)
    hs_list = []
    for k in range(NL):
        Wk_full = lre_msg_W[k]
        Wc = jnp.concatenate([Wk_full[0:D_], Wk_full[D_:2 * D_],
                              Wk_full[3 * D_:4 * D_], Wk_full[4 * D_:5 * D_]], axis=0)
        csum3 = jnp.sum(Wk_full[2 * D_:3 * D_], axis=0).reshape(1, D_)
        bk_row = lre_msg_b[k].reshape(1, D_)
        h_src = _gather_rows(h, srcg)
        wm = _lre_msg(h_src, h_r, conf, gate, src0, Wc, csum3, bk_row)
        aggr = _scatter_add(wm, dstg)
        h = _lre_upd(aggr, zeros_bn, h, lre_upd_W[k],
                     lre_upd_b[k].reshape(1, D_), lre_g, lre_b)
        hs_list.append(h)

    noise = jax.random.normal(jax.random.key(42), (B_, N_, D_)).reshape(BN, D_) * 0.1
    h2 = _add2(dist_emb, noise.astype(f32))
    for k in range(NL):
        h2_src = _gather_rows(h2, srcg)
        wm = _sfe_msg(h2_src, h_r, dist_src, conf, sfe_msg_W[k],
                      sfe_msg_b[k].reshape(1, D_))
        aggr = _scatter_add(wm, dstg)
        h2 = _sfe_upd(aggr, zeros_bn, h2, sfe_upd_W[k],
                      sfe_upd_b[k].reshape(1, D_))

    ctx_all = jnp.stack(
        [hk.reshape(B_, N_, D_)[:, 0, :] for hk in hs_list], axis=1)
    ctx_all = jnp.concatenate(
        [ctx_all, jnp.zeros((B_, 8 - NL, D_), f32)], axis=1)
    return _finale(ctx_all, h2, qr2, rel_table,
                   att_W[0:D_], att_W[D_:2 * D_], att_b.reshape(1, 1),
                   Wq, bq.reshape(1, D_), Wk, bk.reshape(1, D_),
                   Wv, bv.reshape(1, D_), fmr_ln_g.reshape(1, D_),
                   fmr_ln_b.reshape(1, D_))


# speedup vs baseline: 5.5601x; 2.0735x over previous
import functools
import math

import jax
import jax.numpy as jnp
from jax import lax
from jax.experimental import pallas as pl
from jax.experimental.pallas import tpu as pltpu

B_, N_, E_, D_ = 4, 2048, 16384, 64
NR, NL, TAU, M_ = 500, 3, 0.1, 20
BE = B_ * E_
BN = B_ * N_
EC = 2048
NEC = BE // EC
CPB = E_ // EC


def _pre_body(scores_ref, ecm_ref, hr_ref, qr_ref, confB_ref, confW_ref,
              confb_ref, rel_ref, betaW_ref, betab_ref, conf_ref, gate_ref):
    b = pl.program_id(0) // CPB
    s = scores_ref[...]
    m = ecm_ref[...]
    s3 = s * m
    xp = (2.0 * math.pi) * s3 * confB_ref[...]
    cs = jnp.concatenate([jnp.cos(xp), jnp.sin(xp)], axis=1)
    conf_ref[...] = cs @ confW_ref[...] + confb_ref[...]
    rtb = rel_ref[...] @ betaW_ref[...]
    qr = qr_ref[...]
    i500 = lax.broadcasted_iota(jnp.int32, (B_, NR), 1)
    qoh = (qr == i500).astype(jnp.float32)
    rqbw = qoh @ rtb
    i4 = lax.broadcasted_iota(jnp.int32, (B_, 1), 0)
    rqbw_b = jnp.sum(jnp.where(i4 == b, rqbw, 0.0), axis=0, keepdims=True)
    beta = jax.nn.sigmoid(hr_ref[...] @ betaW_ref[...] + rqbw_b + betab_ref[...])
    gate = m * jax.nn.sigmoid((s - beta) / TAU) + (1.0 - m) * 0.5
    gate_ref[...] = gate


def _precompute(scores_f, ecm_f, h_r, query_rels, conf_B, conf_W, conf_b,
                rel_table, beta_W, beta_b):
    full = lambda shape: pl.BlockSpec(shape, lambda i: (0, 0))
    chunk = lambda w: pl.BlockSpec((EC, w), lambda i: (i, 0))
    return pl.pallas_call(
        _pre_body,
        grid=(NEC,),
        in_specs=[chunk(1), chunk(1), chunk(D_), full((B_, 1)),
                  full((1, D_ // 2)), full((D_, D_)), full((1, D_)),
                  full((NR, D_)), full((D_, 1)), full((1, 1))],
        out_specs=[chunk(D_), chunk(1)],
        out_shape=[jax.ShapeDtypeStruct((BE, D_), jnp.float32),
                   jax.ShapeDtypeStruct((BE, 1), jnp.float32)],
    )(scores_f, ecm_f, h_r, query_rels, conf_B, conf_W, conf_b,
      rel_table, beta_W, beta_b)


def _lre_msg_body(hs_ref, hr_ref, cf_ref, gate_ref, src0_ref, Wc_ref,
                  csum_ref, bk_ref, wm_ref):
    hs = hs_ref[...]
    hr = hr_ref[...]
    x = jnp.concatenate([hs * hr, hs, hr, cf_ref[...]], axis=1)
    raw = x @ Wc_ref[...] + src0_ref[...] * csum_ref[...] + bk_ref[...]
    wm_ref[...] = gate_ref[...] * jnp.maximum(raw, 0.0)


def _lre_msg(h_src, h_r, conf, gate, src0, Wc, csum3, bk):
    full = lambda shape: pl.BlockSpec(shape, lambda i: (0, 0))
    chunk = lambda w: pl.BlockSpec((EC, w), lambda i: (i, 0))
    return pl.pallas_call(
        _lre_msg_body,
        grid=(NEC,),
        in_specs=[chunk(D_), chunk(D_), chunk(D_), chunk(1), chunk(1),
                  full((4 * D_, D_)), full((1, D_)), full((1, D_))],
        out_specs=chunk(D_),
        out_shape=jax.ShapeDtypeStruct((BE, D_), jnp.float32),
    )(h_src, h_r, conf, gate, src0, Wc, csum3, bk)


def _sfe_msg_body(hs_ref, hr_ref, ds_ref, cf_ref, Wc_ref, bk_ref, wm_ref):
    hs = hs_ref[...]
    hr = hr_ref[...]
    x = jnp.concatenate([hs * hr, hs, ds_ref[...], hr, cf_ref[...]], axis=1)
    wm_ref[...] = jnp.maximum(x @ Wc_ref[...] + bk_ref[...], 0.0)


def _sfe_msg(h_src, h_r, dist_src, conf, Wc, bk):
    full = lambda shape: pl.BlockSpec(shape, lambda i: (0, 0))
    chunk = lambda w: pl.BlockSpec((EC, w), lambda i: (i, 0))
    return pl.pallas_call(
        _sfe_msg_body,
        grid=(NEC,),
        in_specs=[chunk(D_), chunk(D_), chunk(D_), chunk(D_),
                  full((5 * D_, D_)), full((1, D_))],
        out_specs=chunk(D_),
        out_shape=jax.ShapeDtypeStruct((BE, D_), jnp.float32),
    )(h_src, h_r, dist_src, conf, Wc, bk)


def _ln_rows(x, g, b):
    m = jnp.mean(x, axis=1, keepdims=True)
    v = jnp.mean((x - m) ** 2, axis=1, keepdims=True)
    return (x - m) / jnp.sqrt(v + 1e-5) * g + b


def _lre_upd_body(p0_ref, p1_ref, h_ref, W_ref, b_ref, g_ref, lb_ref, o_ref):
    aggr = p0_ref[...] + p1_ref[...]
    o_ref[...] = _ln_rows(h_ref[...] + aggr @ W_ref[...] + b_ref[...],
                          g_ref[...], lb_ref[...])


def _lre_upd(p0, p1, h, W, b, g, lb):
    full = lambda shape: pl.BlockSpec(shape, lambda: (0, 0))
    return pl.pallas_call(
        _lre_upd_body,
        in_specs=[full((BN, D_)), full((BN, D_)), full((BN, D_)),
                  full((D_, D_)), full((1, D_)), full((1, D_)), full((1, D_))],
        out_specs=full((BN, D_)),
        out_shape=jax.ShapeDtypeStruct((BN, D_), jnp.float32),
    )(p0, p1, h, W, b, g, lb)


def _sfe_upd_body(p0_ref, p1_ref, h_ref, W_ref, b_ref, o_ref):
    aggr = p0_ref[...] + p1_ref[...]
    o_ref[...] = h_ref[...] + aggr @ W_ref[...] + b_ref[...]


def _sfe_upd(p0, p1, h, W, b):
    full = lambda shape: pl.BlockSpec(shape, lambda: (0, 0))
    return pl.pallas_call(
        _sfe_upd_body,
        in_specs=[full((BN, D_)), full((BN, D_)), full((BN, D_)),
                  full((D_, D_)), full((1, D_))],
        out_specs=full((BN, D_)),
        out_shape=jax.ShapeDtypeStruct((BN, D_), jnp.float32),
    )(p0, p1, h, W, b)


def _add_body(a_ref, b_ref, o_ref):
    o_ref[...] = a_ref[...] + b_ref[...]


def _add2(a, b):
    full = pl.BlockSpec((BN, D_), lambda: (0, 0))
    return pl.pallas_call(
        _add_body,
        in_specs=[full, full],
        out_specs=full,
        out_shape=jax.ShapeDtypeStruct((BN, D_), jnp.float32),
    )(a, b)


def _finale_body(ctx_ref, h2f_ref, qr_ref, rel_ref,
                 attW1_ref, attW2_ref, attb_ref, Wq_ref, bq_ref, Wk_ref,
                 bk_ref, Wv_ref, bv_ref, g_ref, lb_ref, o_ref):
    b = pl.program_id(0)
    h2f = h2f_ref[...]
    t_state = h2f[0:1, :]
    qr = qr_ref[...]
    i4 = lax.broadcasted_iota(jnp.int32, (B_, 1), 0)
    qr_b = jnp.sum(jnp.where(i4 == b, qr, 0), axis=0, keepdims=True)
    i500c = lax.broadcasted_iota(jnp.int32, (1, NR), 1)
    qoh = (i500c == qr_b).astype(jnp.float32)
    rq = qoh @ rel_ref[...]
    att = h2f @ attW1_ref[...] + (rq @ attW2_ref[...] + attb_ref[...])
    att = jnp.where(att >= 0.0, att, 0.01 * att)
    amax = jnp.max(att, axis=0, keepdims=True)
    ex = jnp.exp(att - amax)
    alpha = ex / jnp.sum(ex, axis=0, keepdims=True)
    iota = lax.broadcasted_iota(jnp.int32, (N_, 1), 0)
    acur = alpha
    rows = []
    for _ in range(M_):
        v = jnp.max(acur, axis=0, keepdims=True)
        eq = acur == v
        fidx = jnp.min(jnp.where(eq, iota, N_), axis=0, keepdims=True)
        sel = iota == fidx
        ohf = sel.astype(jnp.float32)
        rows.append(jnp.sum(ohf * h2f, axis=0, keepdims=True) * v)
        acur = jnp.where(sel, -1.0, acur)
    ctx = ctx_ref[...].reshape(8, D_)[0:NL, :]
    x = jnp.concatenate([ctx] + rows, axis=0)
    Nt = NL + M_
    hd = lax.broadcasted_iota(jnp.int32, (D_, D_), 0) // 16
    hD = lax.broadcasted_iota(jnp.int32, (D_, D_), 1) // 16
    blockones = (hd == hD).astype(jnp.float32)
    q = x @ Wq_ref[...] + bq_ref[...]
    k_ = x @ Wk_ref[...] + bk_ref[...]
    v_ = x @ Wv_ref[...] + bv_ref[...]

    def nrmh(t):
        ssum = (t * t) @ blockones
        return t / jnp.maximum(jnp.sqrt(ssum), 1e-12)

    q = nrmh(q)
    k_ = nrmh(k_)
    KtV = lax.dot_general(k_, v_, (((0,), (0,)), ((), ())))
    kvs = KtV * blockones
    vsum = jnp.sum(v_, axis=0, keepdims=True)
    ksum = jnp.sum(k_, axis=0, keepdims=True)
    num = q @ kvs + vsum + v_ * float(Nt)
    den = (q * ksum) @ blockones + float(2 * Nt)
    out = num / den
    y = _ln_rows(x + out, g_ref[...], lb_ref[...])
    res = jnp.mean(y, axis=0, keepdims=True) + t_state
    o_ref[...] = jnp.concatenate(
        [res, jnp.zeros((7, D_), jnp.float32)], axis=0).reshape(1, 8, D_)


def _finale(ctx_all, h2f, query_rels, rel_table, attW1, attW2, attb,
            Wq, bq, Wk, bk, Wv, bv, g, lb):
    full = lambda shape: pl.BlockSpec(shape, lambda b: tuple(0 for _ in shape))
    return pl.pallas_call(
        _finale_body,
        grid=(B_,),
        in_specs=[pl.BlockSpec((1, 8, D_), lambda b: (b, 0, 0)),
                  pl.BlockSpec((N_, D_), lambda b: (b, 0)),
                  full((B_, 1)), full((NR, D_)),
                  full((D_, 1)), full((D_, 1)), full((1, 1)),
                  full((D_, D_)), full((1, D_)), full((D_, D_)), full((1, D_)),
                  full((D_, D_)), full((1, D_)), full((1, D_)), full((1, D_))],
        out_specs=pl.BlockSpec((1, 8, D_), lambda b: (b, 0, 0)),
        out_shape=jax.ShapeDtypeStruct((B_, 8, D_), jnp.float32),
    )(ctx_all, h2f, query_rels, rel_table, attW1, attW2, attb,
      Wq, bq, Wk, bk, Wv, bv, g, lb)[:, 0, :]


NC, NS = 2, 16
NW = NC * NS


def _sc_mesh():
    from jax.experimental.pallas import tpu_sc as plsc
    return plsc.VectorSubcoreMesh(core_axis_name="c", subcore_axis_name="s")


def _gather_rows(table, idx, chunk=512):
    n = idx.shape[0]
    D = table.shape[1]
    per_w = n // NW
    c = min(chunk, per_w)
    nch = per_w // c

    @functools.partial(
        pl.kernel,
        out_type=jax.ShapeDtypeStruct((n, D), jnp.float32),
        mesh=_sc_mesh(),
        scratch_types=[pltpu.VMEM((c,), jnp.int32),
                       pltpu.VMEM((c, D), jnp.float32),
                       pltpu.SemaphoreType.DMA],
        compiler_params=pltpu.CompilerParams(use_tc_tiling_on_sc=False),
    )
    def gk(table_hbm, idx_hbm, out_hbm, idx_v, rows_v, sem):
        w = lax.axis_index("c") * NS + lax.axis_index("s")
        base = w * per_w
        for j in range(nch):
            off = base + j * c
            pltpu.sync_copy(idx_hbm.at[pl.ds(off, c)], idx_v)
            pltpu.async_copy(table_hbm.at[idx_v], rows_v, sem).wait()
            pltpu.sync_copy(rows_v, out_hbm.at[pl.ds(off, c)])

    return gk(table, idx)


def _scatter_add_parts(vals, idx3, zeros_bn):
    KCH = E_ * B_ // NW // 128
    RPS = BN // NS

    @functools.partial(
        pl.kernel,
        out_type=jax.ShapeDtypeStruct((NC, BN, D_), jnp.float32),
        mesh=_sc_mesh(),
        scratch_types=[pltpu.VMEM((KCH, 128), jnp.int32),
                       pltpu.VMEM((512, D_), jnp.float32),
                       pltpu.VMEM_SHARED((BN, D_), jnp.float32)],
        compiler_params=pltpu.CompilerParams(use_tc_tiling_on_sc=False),
    )
    def sk(vals_hbm, idx_hbm, zeros_hbm, out_hbm, idx_v, vals_v, acc):
        from jax.experimental.pallas import tpu_sc as plsc
        cid = lax.axis_index("c")
        sid = lax.axis_index("s")
        w = cid * NS + sid
        pltpu.sync_copy(zeros_hbm.at[pl.ds(sid * RPS, RPS)],
                        acc.at[pl.ds(sid * RPS, RPS)])
        pltpu.sync_copy(idx_hbm.at[w], idx_v)
        plsc.subcore_barrier()
        base = w * (KCH * 128)
        for cch in range(4):
            pltpu.sync_copy(vals_hbm.at[pl.ds(base + cch * 512, 512)], vals_v)
            for j in range(4):
                pltpu.sync_copy(vals_v.at[pl.ds(j * 128, 128)],
                                acc.at[idx_v.at[cch * 4 + j]], add=True)
        plsc.subcore_barrier()
        pltpu.sync_copy(acc.at[pl.ds(sid * RPS, RPS)],
                        out_hbm.at[cid, pl.ds(sid * RPS, RPS)])

    return sk(vals, idx3, zeros_bn)


def kernel(edge_index, rels, dists, query_rels, edge_conf_mask, edge_mask,
           node_mask, scores, conf_B, conf_W, conf_b, rel_table, lre_beta_W,
           lre_beta_b, lre_msg_W, lre_msg_b, lre_upd_W, lre_upd_b, lre_ln_g,
           lre_ln_b, dist_table, sfe_msg_W, sfe_msg_b, sfe_upd_W, sfe_upd_b,
           att_W, att_b, Wq, bq, Wk, bk, Wv, bv, fmr_ln_g, fmr_ln_b):
    f32 = jnp.float32
    src = edge_index[:, 0, :].astype(jnp.int32).reshape(BE)
    dst = edge_index[:, 1, :].astype(jnp.int32).reshape(BE)
    boff = jnp.repeat(jnp.arange(B_, dtype=jnp.int32) * N_, E_)
    srcg = src + boff
    dstg = dst + boff
    rels_f = rels.astype(jnp.int32).reshape(BE)
    dclip = jnp.clip(dists, 0, 9).astype(jnp.int32).reshape(BN)
    scores_f = scores.astype(f32).reshape(BE, 1)
    ecm_f = edge_conf_mask.astype(f32).reshape(BE, 1)
    src0 = (src == 0).astype(f32).reshape(BE, 1)
    qr2 = query_rels.astype(jnp.int32).reshape(B_, 1)
    conf_b2 = conf_b.reshape(1, D_)
    beta_b2 = lre_beta_b.reshape(1, 1)
    zeros_bn = jnp.zeros((BN, D_), f32)
    idx3 = dstg.reshape(NW, BE // NW // 128, 128)

    h_r = _gather_rows(rel_table, rels_f)
    dist_emb = _gather_rows(dist_table, dclip)
    dist_src = _gather_rows(dist_emb, srcg)

    conf, gate = _precompute(scores_f, ecm_f, h_r, qr2, conf_B, conf_W,
                             conf_b2, rel_table, lre_beta_W, beta_b2)

    lre_g = lre_ln_g.reshape(1, D_)
    lre_b = lre_ln_b.reshape(1, D_)
    h = jnp.zeros((BN, D_), f32).at[jnp.arange(B_) * N_].set(1.0)
    hs_list = []
    for k in range(NL):
        Wk_full = lre_msg_W[k]
        Wc = jnp.concatenate([Wk_full[0:D_], Wk_full[D_:2 * D_],
                              Wk_full[3 * D_:4 * D_], Wk_full[4 * D_:5 * D_]], axis=0)
        csum3 = jnp.sum(Wk_full[2 * D_:3 * D_], axis=0).reshape(1, D_)
        bk_row = lre_msg_b[k].reshape(1, D_)
        h_src = _gather_rows(h, srcg)
        wm = _lre_msg(h_src, h_r, conf, gate, src0, Wc, csum3, bk_row)
        parts = _scatter_add_parts(wm, idx3, zeros_bn)
        h = _lre_upd(parts[0], parts[1], h, lre_upd_W[k],
                     lre_upd_b[k].reshape(1, D_), lre_g, lre_b)
        hs_list.append(h)

    noise = jax.random.normal(jax.random.key(42), (B_, N_, D_)).reshape(BN, D_) * 0.1
    h2 = _add2(dist_emb, noise.astype(f32))
    for k in range(NL):
        h2_src = _gather_rows(h2, srcg)
        wm = _sfe_msg(h2_src, h_r, dist_src, conf, sfe_msg_W[k],
                      sfe_msg_b[k].reshape(1, D_))
        parts = _scatter_add_parts(wm, idx3, zeros_bn)
        h2 = _sfe_upd(parts[0], parts[1], h2, sfe_upd_W[k],
                      sfe_upd_b[k].reshape(1, D_))

    ctx_all = jnp.stack(
        [hk.reshape(B_, N_, D_)[:, 0, :] for hk in hs_list], axis=1)
    ctx_all = jnp.concatenate(
        [ctx_all, jnp.zeros((B_, 8 - NL, D_), f32)], axis=1)
    return _finale(ctx_all, h2, qr2, rel_table,
                   att_W[0:D_], att_W[D_:2 * D_], att_b.reshape(1, 1),
                   Wq, bq.reshape(1, D_), Wk, bk.reshape(1, D_),
                   Wv, bv.reshape(1, D_), fmr_ln_g.reshape(1, D_),
                   fmr_ln_b.reshape(1, D_))


# speedup vs baseline: 5.8797x; 1.0575x over previous
import functools
import math

import jax
import jax.numpy as jnp
from jax import lax
from jax.experimental import pallas as pl
from jax.experimental.pallas import tpu as pltpu

B_, N_, E_, D_ = 4, 2048, 16384, 64
NR, NL, TAU, M_ = 500, 3, 0.1, 20
BE = B_ * E_
BN = B_ * N_
EC = 2048
NEC = BE // EC
CPB = E_ // EC


def _pre_body(scores_ref, ecm_ref, hr_ref, qr_ref, confB_ref, confW_ref,
              confb_ref, rel_ref, betaW_ref, betab_ref, conf_ref, gate_ref):
    b = pl.program_id(0) // CPB
    s = scores_ref[...]
    m = ecm_ref[...]
    s3 = s * m
    xp = (2.0 * math.pi) * s3 * confB_ref[...]
    cs = jnp.concatenate([jnp.cos(xp), jnp.sin(xp)], axis=1)
    conf_ref[...] = cs @ confW_ref[...] + confb_ref[...]
    rtb = rel_ref[...] @ betaW_ref[...]
    qr = qr_ref[...]
    i500 = lax.broadcasted_iota(jnp.int32, (B_, NR), 1)
    qoh = (qr == i500).astype(jnp.float32)
    rqbw = qoh @ rtb
    i4 = lax.broadcasted_iota(jnp.int32, (B_, 1), 0)
    rqbw_b = jnp.sum(jnp.where(i4 == b, rqbw, 0.0), axis=0, keepdims=True)
    beta = jax.nn.sigmoid(hr_ref[...] @ betaW_ref[...] + rqbw_b + betab_ref[...])
    gate = m * jax.nn.sigmoid((s - beta) / TAU) + (1.0 - m) * 0.5
    gate_ref[...] = gate


def _precompute(scores_f, ecm_f, h_r, query_rels, conf_B, conf_W, conf_b,
                rel_table, beta_W, beta_b):
    full = lambda shape: pl.BlockSpec(shape, lambda i: (0, 0))
    chunk = lambda w: pl.BlockSpec((EC, w), lambda i: (i, 0))
    return pl.pallas_call(
        _pre_body,
        grid=(NEC,),
        in_specs=[chunk(1), chunk(1), chunk(D_), full((B_, 1)),
                  full((1, D_ // 2)), full((D_, D_)), full((1, D_)),
                  full((NR, D_)), full((D_, 1)), full((1, 1))],
        out_specs=[chunk(D_), chunk(1)],
        out_shape=[jax.ShapeDtypeStruct((BE, D_), jnp.float32),
                   jax.ShapeDtypeStruct((BE, 1), jnp.float32)],
    )(scores_f, ecm_f, h_r, query_rels, conf_B, conf_W, conf_b,
      rel_table, beta_W, beta_b)


def _lre_msg_body(hs_ref, hr_ref, cf_ref, gate_ref, src0_ref, Wc_ref,
                  csum_ref, bk_ref, wm_ref):
    hs = hs_ref[...]
    hr = hr_ref[...]
    x = jnp.concatenate([hs * hr, hs, hr, cf_ref[...]], axis=1)
    raw = x @ Wc_ref[...] + src0_ref[...] * csum_ref[...] + bk_ref[...]
    wm_ref[...] = gate_ref[...] * jnp.maximum(raw, 0.0)


def _lre_msg(h_src, h_r, conf, gate, src0, Wc, csum3, bk):
    full = lambda shape: pl.BlockSpec(shape, lambda i: (0, 0))
    chunk = lambda w: pl.BlockSpec((EC, w), lambda i: (i, 0))
    return pl.pallas_call(
        _lre_msg_body,
        grid=(NEC,),
        in_specs=[chunk(D_), chunk(D_), chunk(D_), chunk(1), chunk(1),
                  full((4 * D_, D_)), full((1, D_)), full((1, D_))],
        out_specs=chunk(D_),
        out_shape=jax.ShapeDtypeStruct((BE, D_), jnp.float32),
    )(h_src, h_r, conf, gate, src0, Wc, csum3, bk)


def _sfe_msg_body(hs_ref, hr_ref, ds_ref, cf_ref, Wc_ref, bk_ref, wm_ref):
    hs = hs_ref[...]
    hr = hr_ref[...]
    x = jnp.concatenate([hs * hr, hs, ds_ref[...], hr, cf_ref[...]], axis=1)
    wm_ref[...] = jnp.maximum(x @ Wc_ref[...] + bk_ref[...], 0.0)


def _sfe_msg(h_src, h_r, dist_src, conf, Wc, bk):
    full = lambda shape: pl.BlockSpec(shape, lambda i: (0, 0))
    chunk = lambda w: pl.BlockSpec((EC, w), lambda i: (i, 0))
    return pl.pallas_call(
        _sfe_msg_body,
        grid=(NEC,),
        in_specs=[chunk(D_), chunk(D_), chunk(D_), chunk(D_),
                  full((5 * D_, D_)), full((1, D_))],
        out_specs=chunk(D_),
        out_shape=jax.ShapeDtypeStruct((BE, D_), jnp.float32),
    )(h_src, h_r, dist_src, conf, Wc, bk)


def _ln_rows(x, g, b):
    m = jnp.mean(x, axis=1, keepdims=True)
    v = jnp.mean((x - m) ** 2, axis=1, keepdims=True)
    return (x - m) / jnp.sqrt(v + 1e-5) * g + b


def _lre_upd_body(p0_ref, h_ref, W_ref, b_ref, g_ref, lb_ref, o_ref):
    aggr = p0_ref[...]
    o_ref[...] = _ln_rows(h_ref[...] + aggr @ W_ref[...] + b_ref[...],
                          g_ref[...], lb_ref[...])


def _lre_upd(p0, h, W, b, g, lb):
    full = lambda shape: pl.BlockSpec(shape, lambda: (0, 0))
    return pl.pallas_call(
        _lre_upd_body,
        in_specs=[full((BN, D_)), full((BN, D_)),
                  full((D_, D_)), full((1, D_)), full((1, D_)), full((1, D_))],
        out_specs=full((BN, D_)),
        out_shape=jax.ShapeDtypeStruct((BN, D_), jnp.float32),
    )(p0, h, W, b, g, lb)


def _sfe_upd_body(p0_ref, h_ref, W_ref, b_ref, o_ref):
    o_ref[...] = h_ref[...] + p0_ref[...] @ W_ref[...] + b_ref[...]


def _sfe_upd(p0, h, W, b):
    full = lambda shape: pl.BlockSpec(shape, lambda: (0, 0))
    return pl.pallas_call(
        _sfe_upd_body,
        in_specs=[full((BN, D_)), full((BN, D_)),
                  full((D_, D_)), full((1, D_))],
        out_specs=full((BN, D_)),
        out_shape=jax.ShapeDtypeStruct((BN, D_), jnp.float32),
    )(p0, h, W, b)


def _add_body(a_ref, b_ref, o_ref):
    o_ref[...] = a_ref[...] + b_ref[...]


def _add2(a, b):
    full = pl.BlockSpec((BN, D_), lambda: (0, 0))
    return pl.pallas_call(
        _add_body,
        in_specs=[full, full],
        out_specs=full,
        out_shape=jax.ShapeDtypeStruct((BN, D_), jnp.float32),
    )(a, b)


def _finale_body(ctx_ref, h2f_ref, qr_ref, rel_ref,
                 attW1_ref, attW2_ref, attb_ref, Wq_ref, bq_ref, Wk_ref,
                 bk_ref, Wv_ref, bv_ref, g_ref, lb_ref, o_ref):
    b = pl.program_id(0)
    h2f = h2f_ref[...]
    t_state = h2f[0:1, :]
    qr = qr_ref[...]
    i4 = lax.broadcasted_iota(jnp.int32, (B_, 1), 0)
    qr_b = jnp.sum(jnp.where(i4 == b, qr, 0), axis=0, keepdims=True)
    i500c = lax.broadcasted_iota(jnp.int32, (1, NR), 1)
    qoh = (i500c == qr_b).astype(jnp.float32)
    rq = qoh @ rel_ref[...]
    att = h2f @ attW1_ref[...] + (rq @ attW2_ref[...] + attb_ref[...])
    att = jnp.where(att >= 0.0, att, 0.01 * att)
    amax = jnp.max(att, axis=0, keepdims=True)
    ex = jnp.exp(att - amax)
    alpha = ex / jnp.sum(ex, axis=0, keepdims=True)
    iota = lax.broadcasted_iota(jnp.int32, (N_, 1), 0)
    acur = alpha
    rows = []
    for _ in range(M_):
        v = jnp.max(acur, axis=0, keepdims=True)
        eq = acur == v
        fidx = jnp.min(jnp.where(eq, iota, N_), axis=0, keepdims=True)
        sel = iota == fidx
        ohf = sel.astype(jnp.float32)
        rows.append(jnp.sum(ohf * h2f, axis=0, keepdims=True) * v)
        acur = jnp.where(sel, -1.0, acur)
    ctx = ctx_ref[...].reshape(8, D_)[0:NL, :]
    x = jnp.concatenate([ctx] + rows, axis=0)
    Nt = NL + M_
    hd = lax.broadcasted_iota(jnp.int32, (D_, D_), 0) // 16
    hD = lax.broadcasted_iota(jnp.int32, (D_, D_), 1) // 16
    blockones = (hd == hD).astype(jnp.float32)
    q = x @ Wq_ref[...] + bq_ref[...]
    k_ = x @ Wk_ref[...] + bk_ref[...]
    v_ = x @ Wv_ref[...] + bv_ref[...]

    def nrmh(t):
        ssum = (t * t) @ blockones
        return t / jnp.maximum(jnp.sqrt(ssum), 1e-12)

    q = nrmh(q)
    k_ = nrmh(k_)
    KtV = lax.dot_general(k_, v_, (((0,), (0,)), ((), ())))
    kvs = KtV * blockones
    vsum = jnp.sum(v_, axis=0, keepdims=True)
    ksum = jnp.sum(k_, axis=0, keepdims=True)
    num = q @ kvs + vsum + v_ * float(Nt)
    den = (q * ksum) @ blockones + float(2 * Nt)
    out = num / den
    y = _ln_rows(x + out, g_ref[...], lb_ref[...])
    res = jnp.mean(y, axis=0, keepdims=True) + t_state
    o_ref[...] = jnp.concatenate(
        [res, jnp.zeros((7, D_), jnp.float32)], axis=0).reshape(1, 8, D_)


def _finale(ctx_all, h2f, query_rels, rel_table, attW1, attW2, attb,
            Wq, bq, Wk, bk, Wv, bv, g, lb):
    full = lambda shape: pl.BlockSpec(shape, lambda b: tuple(0 for _ in shape))
    return pl.pallas_call(
        _finale_body,
        grid=(B_,),
        in_specs=[pl.BlockSpec((1, 8, D_), lambda b: (b, 0, 0)),
                  pl.BlockSpec((N_, D_), lambda b: (b, 0)),
                  full((B_, 1)), full((NR, D_)),
                  full((D_, 1)), full((D_, 1)), full((1, 1)),
                  full((D_, D_)), full((1, D_)), full((D_, D_)), full((1, D_)),
                  full((D_, D_)), full((1, D_)), full((1, D_)), full((1, D_))],
        out_specs=pl.BlockSpec((1, 8, D_), lambda b: (b, 0, 0)),
        out_shape=jax.ShapeDtypeStruct((B_, 8, D_), jnp.float32),
    )(ctx_all, h2f, query_rels, rel_table, attW1, attW2, attb,
      Wq, bq, Wk, bk, Wv, bv, g, lb)[:, 0, :]


NC, NS = 2, 16
NW = NC * NS


def _sc_mesh():
    from jax.experimental.pallas import tpu_sc as plsc
    return plsc.VectorSubcoreMesh(core_axis_name="c", subcore_axis_name="s")


def _gather_rows(table, idx, chunk=512):
    n = idx.shape[0]
    D = table.shape[1]
    per_w = n // NW
    c = min(chunk, per_w)
    nch = per_w // c

    @functools.partial(
        pl.kernel,
        out_type=jax.ShapeDtypeStruct((n, D), jnp.float32),
        mesh=_sc_mesh(),
        scratch_types=[pltpu.VMEM((per_w,), jnp.int32),
                       pltpu.VMEM((2, c, D), jnp.float32),
                       pltpu.SemaphoreType.DMA, pltpu.SemaphoreType.DMA,
                       pltpu.SemaphoreType.DMA, pltpu.SemaphoreType.DMA],
        compiler_params=pltpu.CompilerParams(use_tc_tiling_on_sc=False),
    )
    def gk(table_hbm, idx_hbm, out_hbm, idx_v, rows_v, g0, g1, w0, w1):
        w = lax.axis_index("c") * NS + lax.axis_index("s")
        base = w * per_w
        pltpu.sync_copy(idx_hbm.at[pl.ds(base, per_w)], idx_v)
        gsem = [g0, g1]
        wsem = [w0, w1]
        gd = [None, None]
        wd = [None, None]
        gd[0] = pltpu.async_copy(table_hbm.at[idx_v.at[pl.ds(0, c)]],
                                 rows_v.at[0], gsem[0])
        for j in range(nch):
            b = j % 2
            nb = (j + 1) % 2
            if j + 1 < nch:
                if wd[nb] is not None:
                    wd[nb].wait()
                gd[nb] = pltpu.async_copy(
                    table_hbm.at[idx_v.at[pl.ds((j + 1) * c, c)]],
                    rows_v.at[nb], gsem[nb])
            gd[b].wait()
            wd[b] = pltpu.async_copy(rows_v.at[b],
                                     out_hbm.at[pl.ds(base + j * c, c)], wsem[b])
        for d in wd:
            if d is not None:
                d.wait()

    return gk(table, idx)


def _scatter_add_bn(vals, idx3, zeros_half):
    KCH = E_ * B_ // NW // 128
    HALF = BN // NC
    RPS = HALF // NS

    @functools.partial(
        pl.kernel,
        out_type=jax.ShapeDtypeStruct((BN, D_), jnp.float32),
        mesh=_sc_mesh(),
        scratch_types=[pltpu.VMEM((KCH, 128), jnp.int32),
                       pltpu.VMEM((2, 512, D_), jnp.float32),
                       pltpu.VMEM_SHARED((HALF, D_), jnp.float32),
                       pltpu.SemaphoreType.DMA, pltpu.SemaphoreType.DMA,
                       pltpu.SemaphoreType.DMA, pltpu.SemaphoreType.DMA],
        compiler_params=pltpu.CompilerParams(use_tc_tiling_on_sc=False),
    )
    def sk(vals_hbm, idx_hbm, zeros_hbm, out_hbm, idx_v, vals_v, acc,
           l0, l1, s0, s1):
        from jax.experimental.pallas import tpu_sc as plsc
        cid = lax.axis_index("c")
        sid = lax.axis_index("s")
        w = cid * NS + sid
        pltpu.sync_copy(zeros_hbm.at[pl.ds(sid * RPS, RPS)],
                        acc.at[pl.ds(sid * RPS, RPS)])
        pltpu.sync_copy(idx_hbm.at[w], idx_v)
        plsc.subcore_barrier()
        base = w * (KCH * 128)
        lsem = [l0, l1]
        ssem = [s0, s1]
        ld = [None, None]
        sd = [[], []]
        ld[0] = pltpu.async_copy(vals_hbm.at[pl.ds(base, 512)],
                                 vals_v.at[0], lsem[0])
        for j in range(4):
            b = j % 2
            nb = (j + 1) % 2
            if j + 1 < 4:
                for d in sd[nb]:
                    d.wait()
                sd[nb] = []
                ld[nb] = pltpu.async_copy(
                    vals_hbm.at[pl.ds(base + (j + 1) * 512, 512)],
                    vals_v.at[nb], lsem[nb])
            ld[b].wait()
            sd[b] = [pltpu.async_copy(vals_v.at[b].at[pl.ds(t * 128, 128)],
                                      acc.at[idx_v.at[j * 4 + t]], ssem[b],
                                      add=True)
                     for t in range(4)]
        for bb in (0, 1):
            for d in sd[bb]:
                d.wait()
        plsc.subcore_barrier()
        pltpu.sync_copy(acc.at[pl.ds(sid * RPS, RPS)],
                        out_hbm.at[pl.ds(cid * HALF + sid * RPS, RPS)])

    return sk(vals, idx3, zeros_half)


def kernel(edge_index, rels, dists, query_rels, edge_conf_mask, edge_mask,
           node_mask, scores, conf_B, conf_W, conf_b, rel_table, lre_beta_W,
           lre_beta_b, lre_msg_W, lre_msg_b, lre_upd_W, lre_upd_b, lre_ln_g,
           lre_ln_b, dist_table, sfe_msg_W, sfe_msg_b, sfe_upd_W, sfe_upd_b,
           att_W, att_b, Wq, bq, Wk, bk, Wv, bv, fmr_ln_g, fmr_ln_b):
    f32 = jnp.float32
    src = edge_index[:, 0, :].astype(jnp.int32).reshape(BE)
    dst = edge_index[:, 1, :].astype(jnp.int32).reshape(BE)
    boff = jnp.repeat(jnp.arange(B_, dtype=jnp.int32) * N_, E_)
    srcg = src + boff
    dstg = dst + boff
    rels_f = rels.astype(jnp.int32).reshape(BE)
    dclip = jnp.clip(dists, 0, 9).astype(jnp.int32).reshape(BN)
    scores_f = scores.astype(f32).reshape(BE, 1)
    ecm_f = edge_conf_mask.astype(f32).reshape(BE, 1)
    src0 = (src == 0).astype(f32).reshape(BE, 1)
    qr2 = query_rels.astype(jnp.int32).reshape(B_, 1)
    conf_b2 = conf_b.reshape(1, D_)
    beta_b2 = lre_beta_b.reshape(1, 1)
    zeros_half = jnp.zeros((BN // NC, D_), f32)
    idx3 = dstg.reshape(NW, BE // NW // 128, 128)
    idx3 = idx3 - (jnp.arange(NW, dtype=jnp.int32)[:, None, None] // NS) * (BN // NC)

    h_r = _gather_rows(rel_table, rels_f)
    dist_emb = _gather_rows(dist_table, dclip)
    dist_src = _gather_rows(dist_emb, srcg)

    conf, gate = _precompute(scores_f, ecm_f, h_r, qr2, conf_B, conf_W,
                             conf_b2, rel_table, lre_beta_W, beta_b2)

    lre_g = lre_ln_g.reshape(1, D_)
    lre_b = lre_ln_b.reshape(1, D_)
    h = jnp.zeros((BN, D_), f32).at[jnp.arange(B_) * N_].set(1.0)
    hs_list = []
    for k in range(NL):
        Wk_full = lre_msg_W[k]
        Wc = jnp.concatenate([Wk_full[0:D_], Wk_full[D_:2 * D_],
                              Wk_full[3 * D_:4 * D_], Wk_full[4 * D_:5 * D_]], axis=0)
        csum3 = jnp.sum(Wk_full[2 * D_:3 * D_], axis=0).reshape(1, D_)
        bk_row = lre_msg_b[k].reshape(1, D_)
        h_src = _gather_rows(h, srcg)
        wm = _lre_msg(h_src, h_r, conf, gate, src0, Wc, csum3, bk_row)
        aggr = _scatter_add_bn(wm, idx3, zeros_half)
        h = _lre_upd(aggr, h, lre_upd_W[k],
                     lre_upd_b[k].reshape(1, D_), lre_g, lre_b)
        hs_list.append(h)

    noise = jax.random.normal(jax.random.key(42), (B_, N_, D_)).reshape(BN, D_) * 0.1
    h2 = _add2(dist_emb, noise.astype(f32))
    for k in range(NL):
        h2_src = _gather_rows(h2, srcg)
        wm = _sfe_msg(h2_src, h_r, dist_src, conf, sfe_msg_W[k],
                      sfe_msg_b[k].reshape(1, D_))
        aggr = _scatter_add_bn(wm, idx3, zeros_half)
        h2 = _sfe_upd(aggr, h2, sfe_upd_W[k],
                      sfe_upd_b[k].reshape(1, D_))

    ctx_all = jnp.stack(
        [hk.reshape(B_, N_, D_)[:, 0, :] for hk in hs_list], axis=1)
    ctx_all = jnp.concatenate(
        [ctx_all, jnp.zeros((B_, 8 - NL, D_), f32)], axis=1)
    return _finale(ctx_all, h2, qr2, rel_table,
                   att_W[0:D_], att_W[D_:2 * D_], att_b.reshape(1, 1),
                   Wq, bq.reshape(1, D_), Wk, bk.reshape(1, D_),
                   Wv, bv.reshape(1, D_), fmr_ln_g.reshape(1, D_),
                   fmr_ln_b.reshape(1, D_))
